# Initial kernel scaffold; baseline (speedup 1.0000x reference)
import jax
import jax.numpy as jnp
from jax.experimental import pallas as pl


def kernel(byte_ids, embed_table, W, b):
    raise NotImplementedError("write your pallas kernel here")


# speedup vs baseline: 1.8794x; 1.8794x over previous
import functools

import jax
import jax.numpy as jnp
from jax import lax
from jax.experimental import pallas as pl
from jax.experimental.pallas import tpu as pltpu
from jax.experimental.pallas import tpu_sc as plsc

VOCAB = 256
D_MODEL = 384

_NC = 2
_NS = 16
_NW = _NC * _NS

_N = 4 * 8192
_BPW = _N // _NW
_CH = 128
_NCH = _BPW // _CH


def _table_body(e_ref, w_ref, b_ref, t_ref):
    t_ref[...] = lax.dot_general(
        e_ref[...], w_ref[...],
        dimension_numbers=(((1,), (1,)), ((), ())),
        preferred_element_type=jnp.float32,
    ) + b_ref[...]


_fuse_table = pl.pallas_call(
    _table_body,
    out_shape=jax.ShapeDtypeStruct((VOCAB, D_MODEL), jnp.float32),
)


def _gather_body(ids_hbm, table_hbm, out_hbm, idx_v, rows_v, g0, g1, s0, s1):
    wid = lax.axis_index("s") * _NC + lax.axis_index("c")
    base = wid * _BPW

    pltpu.sync_copy(ids_hbm.at[pl.ds(base, _BPW)], idx_v)
    for i in range(_BPW // 16):
        sl = pl.ds(i * 16, 16)
        idx_v[sl] = jnp.clip(idx_v[sl], 0, VOCAB - 1)

    gsems = (g0, g1)
    ssems = (s0, s1)
    gh = [None, None]
    sh = [None, None]

    def start_gather(c):
        buf = c & 1
        if sh[buf] is not None:
            sh[buf].wait()
        gh[buf] = pltpu.async_copy(
            table_hbm.at[idx_v.at[pl.ds(c * _CH, _CH)]],
            rows_v.at[buf], gsems[buf])

    start_gather(0)
    for c in range(_NCH):
        buf = c & 1
        gh[buf].wait()
        sh[buf] = pltpu.async_copy(
            rows_v.at[buf],
            out_hbm.at[pl.ds(base + c * _CH, _CH)], ssems[buf])
        if c + 1 < _NCH:
            start_gather(c + 1)
    sh[0].wait()
    sh[1].wait()


_gather = pl.kernel(
    _gather_body,
    out_type=jax.ShapeDtypeStruct((_N, D_MODEL), jnp.float32),
    mesh=plsc.VectorSubcoreMesh(core_axis_name="c", subcore_axis_name="s"),
    scratch_types=[
        pltpu.VMEM((_BPW,), jnp.int32),
        pltpu.VMEM((2, _CH, D_MODEL), jnp.float32),
        pltpu.SemaphoreType.DMA,
        pltpu.SemaphoreType.DMA,
        pltpu.SemaphoreType.DMA,
        pltpu.SemaphoreType.DMA,
    ],
)


@jax.jit
def kernel(byte_ids, embed_table, W, b):
    table = _fuse_table(embed_table, W, b.reshape(1, D_MODEL))
    ids = byte_ids.reshape(-1)
    out = _gather(ids, table)
    return out.reshape(byte_ids.shape + (D_MODEL,))
